# baseline (speedup 1.0000x reference)
import functools

import jax
import jax.numpy as jnp
from jax import lax
from jax.experimental import pallas as pl
from jax.experimental.pallas import tpu as pltpu
from jax.experimental.pallas import tpu_sc as plsc

L = 16
N = 8192
NV = N // L
B = 32
D = 128
W = 128
NWIN = N // W
NDEPTH = 4
NC = 2

BINS0, BINS1, BINS2 = 2048, 2048, 1024
SEG0, SEG1, SEG2 = 0, 2048, 4096
NHIST = BINS0 + BINS1 + BINS2


@functools.partial(
    pl.kernel,
    out_type=(
        jax.ShapeDtypeStruct((B * N, D), jnp.float32),
        jax.ShapeDtypeStruct((B, N), jnp.float32),
    ),
    mesh=plsc.VectorSubcoreMesh(core_axis_name="c", subcore_axis_name="s"),
    compiler_params=pltpu.CompilerParams(needs_layout_passes=False),
    scratch_types=[
        pltpu.VMEM((N,), jnp.int32),
        pltpu.VMEM((N,), jnp.int32),
        pltpu.VMEM((N,), jnp.int32),
        pltpu.VMEM((N,), jnp.int32),
        pltpu.VMEM((NHIST,), jnp.int32),
        pltpu.VMEM((N,), jnp.float32),
        pltpu.VMEM((NDEPTH, W, D), jnp.float32),
        [pltpu.SemaphoreType.DMA] * NDEPTH,
        [pltpu.SemaphoreType.DMA] * NDEPTH,
    ],
)
def _sc_sort_gather(x_hbm, fit_hbm, y_hbm, fs_hbm,
                    key_v, idx_v, key2_v, idx2_v, hist_v, fit_v, buf_v,
                    sems_in, sems_out):
    b = lax.axis_index("s") * NC + lax.axis_index("c")

    pltpu.sync_copy(fit_hbm.at[b], fit_v)

    def zero_body(g, _):
        hist_v[pl.ds(g * L, L)] = jnp.zeros((L,), jnp.int32)
        return 0
    lax.fori_loop(0, NHIST // L, zero_body, 0, unroll=4)

    def digits(k):
        d0 = k & jnp.int32(BINS0 - 1)
        d1 = lax.shift_right_logical(k, 11) & jnp.int32(BINS1 - 1)
        d2 = lax.shift_right_logical(k, 22)
        return d0, d1 + SEG1, d2 + SEG2

    def count_body(v, _):
        f = fit_v[pl.ds(v * L, L)]
        k = lax.bitcast_convert_type(f, jnp.int32)
        k = k ^ (lax.shift_right_arithmetic(k, 31) | jnp.int32(-0x80000000))
        key_v[pl.ds(v * L, L)] = k
        idx_v[pl.ds(v * L, L)] = lax.iota(jnp.int32, L) + v * L
        for d in digits(k):
            cnt, lm = plsc.scan_count(d)
            plsc.addupdate_scatter(hist_v, [d], cnt, mask=lm)
        return 0
    lax.fori_loop(0, NV, count_body, 0, unroll=2)

    for seg, bins in ((SEG0, BINS0), (SEG1, BINS1), (SEG2, BINS2)):
        def scan_body(g, carry, seg=seg):
            s = pl.ds(seg + g * L, L)
            hv = hist_v[s]
            inc = plsc.cumsum(hv) + carry
            hist_v[s] = inc - hv
            return carry + jnp.sum(hv)
        lax.fori_loop(0, bins // L, scan_body, jnp.int32(0), unroll=False)

    def perm1_body(v, _):
        k = key_v[pl.ds(v * L, L)]
        i = idx_v[pl.ds(v * L, L)]
        d = k & jnp.int32(BINS0 - 1)
        cnt, lm = plsc.scan_count(d)
        pos = plsc.load_gather(hist_v, [d]) + cnt - 1
        plsc.store_scatter(key2_v, [pos], k)
        plsc.store_scatter(idx2_v, [pos], i)
        plsc.addupdate_scatter(hist_v, [d], cnt, mask=lm)
        return 0
    lax.fori_loop(0, NV, perm1_body, 0, unroll=2)

    def perm2_body(v, _):
        k = key2_v[pl.ds(v * L, L)]
        i = idx2_v[pl.ds(v * L, L)]
        d = (lax.shift_right_logical(k, 11) & jnp.int32(BINS1 - 1)) + SEG1
        cnt, lm = plsc.scan_count(d)
        pos = plsc.load_gather(hist_v, [d]) + cnt - 1
        plsc.store_scatter(key_v, [pos], k)
        plsc.store_scatter(idx_v, [pos], i)
        plsc.addupdate_scatter(hist_v, [d], cnt, mask=lm)
        return 0
    lax.fori_loop(0, NV, perm2_body, 0, unroll=2)

    def perm3_body(v, _):
        k = key_v[pl.ds(v * L, L)]
        i = idx_v[pl.ds(v * L, L)]
        d = lax.shift_right_logical(k, 22) + SEG2
        cnt, lm = plsc.scan_count(d)
        pos = plsc.load_gather(hist_v, [d]) + cnt - 1
        m2 = lax.shift_right_arithmetic(k, 31)
        kf = k ^ (jnp.int32(-0x80000000) | (~m2 & jnp.int32(0x7FFFFFFF)))
        plsc.store_scatter(fit_v, [pos], lax.bitcast_convert_type(kf, jnp.float32))
        plsc.store_scatter(idx2_v, [pos], i + b * N)
        plsc.addupdate_scatter(hist_v, [d], cnt, mask=lm)
        return 0
    lax.fori_loop(0, NV, perm3_body, 0, unroll=2)

    pltpu.sync_copy(fit_v, fs_hbm.at[b])

    def in_start(w, slot):
        pltpu.async_copy(x_hbm.at[idx2_v.at[pl.ds(w * W, W)]],
                         buf_v.at[slot], sems_in[slot])

    def in_wait(w, slot):
        pltpu.make_async_copy(x_hbm.at[idx2_v.at[pl.ds(w * W, W)]],
                              buf_v.at[slot], sems_in[slot]).wait()

    def out_start(w, slot):
        pltpu.async_copy(buf_v.at[slot], y_hbm.at[pl.ds(b * N + w * W, W)],
                         sems_out[slot])

    def out_wait(w, slot):
        pltpu.make_async_copy(buf_v.at[slot], y_hbm.at[pl.ds(b * N + w * W, W)],
                              sems_out[slot]).wait()

    for s in range(NDEPTH):
        in_start(s, s)
    for s in range(NDEPTH):
        in_wait(s, s)
        out_start(s, s)

    def gather_body(t, _):
        w0 = NDEPTH * t
        for s in range(NDEPTH):
            out_wait(w0 + s - NDEPTH, s)
            in_start(w0 + s, s)
        for s in range(NDEPTH):
            in_wait(w0 + s, s)
            out_start(w0 + s, s)
        return 0
    lax.fori_loop(1, NWIN // NDEPTH, gather_body, 0, unroll=False)
    for s in range(NDEPTH):
        out_wait(NWIN - NDEPTH + s, s)


def kernel(x, fitness):
    xflat = x.reshape(B * N, D)
    yflat, fit_sorted = _sc_sort_gather(xflat, fitness)
    return yflat.reshape(B, N, D), fit_sorted


# speedup vs baseline: 1.0250x; 1.0250x over previous
import functools

import jax
import jax.numpy as jnp
from jax import lax
from jax.experimental import pallas as pl
from jax.experimental.pallas import tpu as pltpu
from jax.experimental.pallas import tpu_sc as plsc

L = 16
N = 8192
NV = N // L
B = 32
D = 128
W = 64
NWIN = N // W
NDEPTH = 8
NC = 2

BINS0, BINS1, BINS2 = 2048, 2048, 1024
SEG0, SEG1, SEG2 = 0, 2048, 4096
NHIST = BINS0 + BINS1 + BINS2


@functools.partial(
    pl.kernel,
    out_type=(
        jax.ShapeDtypeStruct((B * N, D), jnp.float32),
        jax.ShapeDtypeStruct((B, N), jnp.float32),
    ),
    mesh=plsc.VectorSubcoreMesh(core_axis_name="c", subcore_axis_name="s"),
    compiler_params=pltpu.CompilerParams(needs_layout_passes=False),
    scratch_types=[
        pltpu.VMEM((N,), jnp.int32),
        pltpu.VMEM((N,), jnp.int32),
        pltpu.VMEM((N,), jnp.int32),
        pltpu.VMEM((N,), jnp.int32),
        pltpu.VMEM((NHIST,), jnp.int32),
        pltpu.VMEM((N,), jnp.float32),
        pltpu.VMEM((NDEPTH, W, D), jnp.float32),
        [pltpu.SemaphoreType.DMA] * NDEPTH,
    ],
)
def _sc_sort_gather(x_hbm, fit_hbm, y_hbm, fs_hbm,
                    key_v, idx_v, key2_v, idx2_v, hist_v, fit_v, buf_v, sems):
    b = lax.axis_index("s") * NC + lax.axis_index("c")

    pltpu.sync_copy(fit_hbm.at[b], fit_v)

    def zero_body(g, _):
        hist_v[pl.ds(g * L, L)] = jnp.zeros((L,), jnp.int32)
        return 0
    lax.fori_loop(0, NHIST // L, zero_body, 0, unroll=4)

    def digits(k):
        d0 = k & jnp.int32(BINS0 - 1)
        d1 = lax.shift_right_logical(k, 11) & jnp.int32(BINS1 - 1)
        d2 = lax.shift_right_logical(k, 22)
        return d0, d1 + SEG1, d2 + SEG2

    def count_body(v, _):
        f = fit_v[pl.ds(v * L, L)]
        k = lax.bitcast_convert_type(f, jnp.int32)
        k = k ^ (lax.shift_right_arithmetic(k, 31) | jnp.int32(-0x80000000))
        key_v[pl.ds(v * L, L)] = k
        idx_v[pl.ds(v * L, L)] = lax.iota(jnp.int32, L) + v * L
        for d in digits(k):
            cnt, lm = plsc.scan_count(d)
            plsc.addupdate_scatter(hist_v, [d], cnt, mask=lm)
        return 0
    lax.fori_loop(0, NV, count_body, 0, unroll=2)

    for seg, bins in ((SEG0, BINS0), (SEG1, BINS1), (SEG2, BINS2)):
        def scan_body(g, carry, seg=seg):
            s = pl.ds(seg + g * L, L)
            hv = hist_v[s]
            inc = plsc.cumsum(hv) + carry
            hist_v[s] = inc - hv
            return carry + jnp.sum(hv)
        lax.fori_loop(0, bins // L, scan_body, jnp.int32(0), unroll=False)

    def perm1_body(v, _):
        k = key_v[pl.ds(v * L, L)]
        i = idx_v[pl.ds(v * L, L)]
        d = k & jnp.int32(BINS0 - 1)
        cnt, lm = plsc.scan_count(d)
        pos = plsc.load_gather(hist_v, [d]) + cnt - 1
        plsc.store_scatter(key2_v, [pos], k)
        plsc.store_scatter(idx2_v, [pos], i)
        plsc.addupdate_scatter(hist_v, [d], cnt, mask=lm)
        return 0
    lax.fori_loop(0, NV, perm1_body, 0, unroll=2)

    def perm2_body(v, _):
        k = key2_v[pl.ds(v * L, L)]
        i = idx2_v[pl.ds(v * L, L)]
        d = (lax.shift_right_logical(k, 11) & jnp.int32(BINS1 - 1)) + SEG1
        cnt, lm = plsc.scan_count(d)
        pos = plsc.load_gather(hist_v, [d]) + cnt - 1
        plsc.store_scatter(key_v, [pos], k)
        plsc.store_scatter(idx_v, [pos], i)
        plsc.addupdate_scatter(hist_v, [d], cnt, mask=lm)
        return 0
    lax.fori_loop(0, NV, perm2_body, 0, unroll=2)

    def perm3_body(v, _):
        k = key_v[pl.ds(v * L, L)]
        i = idx_v[pl.ds(v * L, L)]
        d = lax.shift_right_logical(k, 22) + SEG2
        cnt, lm = plsc.scan_count(d)
        pos = plsc.load_gather(hist_v, [d]) + cnt - 1
        m2 = lax.shift_right_arithmetic(k, 31)
        kf = k ^ (jnp.int32(-0x80000000) | (~m2 & jnp.int32(0x7FFFFFFF)))
        plsc.store_scatter(fit_v, [pos], lax.bitcast_convert_type(kf, jnp.float32))
        plsc.store_scatter(idx2_v, [pos], i + b * N)
        plsc.addupdate_scatter(hist_v, [d], cnt, mask=lm)
        return 0
    lax.fori_loop(0, NV, perm3_body, 0, unroll=2)

    pltpu.sync_copy(fit_v, fs_hbm.at[b])

    def g_start(w, slot):
        pltpu.async_copy(x_hbm.at[idx2_v.at[pl.ds(w * W, W)]],
                         buf_v.at[slot], sems[slot])

    def g_wait(w, slot):
        pltpu.make_async_copy(x_hbm.at[idx2_v.at[pl.ds(w * W, W)]],
                              buf_v.at[slot], sems[slot]).wait()

    def g_out(w, slot):
        pltpu.sync_copy(buf_v.at[slot], y_hbm.at[pl.ds(b * N + w * W, W)])

    for s in range(NDEPTH):
        g_start(s, s)

    def gather_body(t, _):
        w0 = NDEPTH * t
        for s in range(NDEPTH):
            g_wait(w0 + s, s)
            g_out(w0 + s, s)
            g_start(w0 + s + NDEPTH, s)
        return 0
    lax.fori_loop(0, NWIN // NDEPTH - 1, gather_body, 0, unroll=False)
    for s in range(NDEPTH):
        g_wait(NWIN - NDEPTH + s, s)
        g_out(NWIN - NDEPTH + s, s)


def kernel(x, fitness):
    xflat = x.reshape(B * N, D)
    yflat, fit_sorted = _sc_sort_gather(xflat, fitness)
    return yflat.reshape(B, N, D), fit_sorted
